# baseline (speedup 1.0000x reference)
import dataclasses

import jax
import jax.numpy as jnp
from jax import lax
from jax.experimental import pallas as pl
from jax.experimental.pallas import tpu as pltpu
from jax.experimental.pallas import tpu_sc as plsc

B = 8
C = 96
H_OUT = 224
W_OUT = 224
P = H_OUT * W_OUT * C
NB = 112 * 112 * C
ROWS_B = NB // 128
WR = 48
W = WR * 128
NWIN = ROWS_B // WR
NCH = 3
CH = P // NCH
SLICE = CH // 16
ZB = 3584
NZ = SLICE // ZB
CHUNKS = B * NCH


def _compiler_params():
    cp = pltpu.CompilerParams()
    if "needs_layout_passes" in pltpu.CompilerParams.__dataclass_fields__:
        cp = dataclasses.replace(cp, needs_layout_passes=False)
    return cp


def kernel(inputs, indices, output_shape):
    rt0_i = output_shape[0].astype(jnp.int32) - 8
    rt0_f = rt0_i.astype(jnp.float32)
    vals = inputs.reshape(-1) + rt0_f
    idx = indices.astype(jnp.int32).reshape(-1) + rt0_i
    mesh = plsc.VectorSubcoreMesh(core_axis_name="c", subcore_axis_name="s")

    @pl.kernel(
        out_type=jax.ShapeDtypeStruct((B * P,), jnp.float32),
        mesh=mesh,
        scratch_types=[
            pltpu.VMEM((W,), jnp.int32),
            pltpu.VMEM((W,), jnp.int32),
            pltpu.VMEM((W,), jnp.float32),
            pltpu.VMEM((W,), jnp.float32),
            pltpu.VMEM((ZB,), jnp.float32),
            pltpu.VMEM_SHARED((CH,), jnp.float32),
            pltpu.SemaphoreType.DMA,
            pltpu.SemaphoreType.DMA,
            pltpu.SemaphoreType.DMA,
            pltpu.SemaphoreType.DMA,
            pltpu.SemaphoreType.DMA,
        ],
        compiler_params=_compiler_params(),
    )
    def scatter_add_kernel(idx_hbm, vals_hbm, out_hbm, ibuf0, ibuf1, vbuf0,
                           vbuf1, zbuf, shared, semi0, semi1, semv0, semv1,
                           sem):
        core = lax.axis_index("c")
        sid = lax.axis_index("s")
        ibufs = (ibuf0, ibuf1)
        vbufs = (vbuf0, vbuf1)
        semis = (semi0, semi1)
        semvs = (semv0, semv1)

        @pl.loop(0, ZB, step=16)
        def _(i):
            zbuf[pl.ds(i, 16)] = jnp.zeros((16,), jnp.float32)

        @pl.loop(0, NZ)
        def _(z):
            pltpu.sync_copy(zbuf, shared.at[pl.ds(sid * SLICE + z * ZB, ZB)])

        def elem_of(b, w):
            return b * NB + (sid + 16 * w) * W

        def issue_load(b, w, k):
            eb = elem_of(b, w)
            pltpu.async_copy(idx_hbm.at[pl.ds(eb, W)], ibufs[k], semis[k])
            pltpu.async_copy(vals_hbm.at[pl.ds(eb, W)], vbufs[k], semvs[k])

        def wait_load(b, w, k):
            eb = elem_of(b, w)
            pltpu.make_async_copy(idx_hbm.at[pl.ds(eb, W)], ibufs[k],
                                  semis[k]).wait()
            pltpu.make_async_copy(vals_hbm.at[pl.ds(eb, W)], vbufs[k],
                                  semvs[k]).wait()

        @pl.loop(0, CHUNKS // 2)
        def _(ci):
            cid = 2 * ci + core
            b = cid // NCH
            j = cid - b * NCH
            lo = j * CH
            nwin = jnp.where(sid < NWIN - 16 * (NWIN // 16), NWIN // 16 + 1,
                             NWIN // 16)
            plsc.subcore_barrier()

            issue_load(b, 0, 0)

            @pl.loop(0, nwin, step=2)
            def _(w0):
                for k in range(2):
                    w = w0 + k

                    @pl.when(w < nwin)
                    def _():
                        @pl.when(w + 1 < nwin)
                        def _():
                            issue_load(b, w + 1, 1 - k)

                        wait_load(b, w, k)
                        ib = ibufs[k]
                        vb = vbufs[k]

                        @pl.loop(0, W, step=128)
                        def _(e0):
                            for cc in range(8):
                                sl = pl.ds(e0 + cc * 16, 16)
                                v = ib[sl]
                                t = v - lo
                                m = (t >= 0) & (t < CH)
                                ib[sl] = jnp.where(m, t, -1)

                        @pl.loop(0, W, step=1024)
                        def _(e0):
                            descs = [
                                pltpu.async_copy(
                                    vb.at[pl.ds(e0 + 128 * k2, 128)],
                                    shared.at[plsc.Indices(
                                        ib.at[pl.ds(e0 + 128 * k2, 128)],
                                        ignored_value=-1)],
                                    sem,
                                    add=True,
                                )
                                for k2 in range(8)
                            ]
                            for d in descs:
                                d.wait()

            plsc.subcore_barrier()
            pltpu.sync_copy(
                shared.at[pl.ds(sid * SLICE, SLICE)],
                out_hbm.at[pl.ds(b * P + lo + sid * SLICE, SLICE)],
            )

            @pl.loop(0, NZ)
            def _(z):
                pltpu.sync_copy(zbuf,
                                shared.at[pl.ds(sid * SLICE + z * ZB, ZB)])

    out = scatter_add_kernel(idx, vals)
    return (out + rt0_f).reshape(B, H_OUT, W_OUT, C)


# speedup vs baseline: 1.1964x; 1.1964x over previous
import dataclasses

import jax
import jax.numpy as jnp
from jax import lax
from jax.experimental import pallas as pl
from jax.experimental.pallas import tpu as pltpu
from jax.experimental.pallas import tpu_sc as plsc

B = 8
C = 96
H_OUT = 224
W_OUT = 224
P = H_OUT * W_OUT * C
NB = 112 * 112 * C
ROWS_B = NB // 128
WR = 32
W = WR * 128
NWIN = ROWS_B // WR
NCH = 3
CH = P // NCH
SLICE = CH // 16
ZB = 3584
NZ = SLICE // ZB
CHUNKS = B * NCH


def _compiler_params():
    cp = pltpu.CompilerParams()
    if "needs_layout_passes" in pltpu.CompilerParams.__dataclass_fields__:
        cp = dataclasses.replace(cp, needs_layout_passes=False)
    return cp


def kernel(inputs, indices, output_shape):
    rt0_i = output_shape[0].astype(jnp.int32) - 8
    rt0_f = rt0_i.astype(jnp.float32)
    vals = inputs.reshape(-1, 128) + rt0_f
    idx = indices.astype(jnp.int32).reshape(-1, 128) + rt0_i
    mesh = plsc.VectorSubcoreMesh(core_axis_name="c", subcore_axis_name="s")

    @pl.kernel(
        out_type=jax.ShapeDtypeStruct((B * P,), jnp.float32),
        mesh=mesh,
        scratch_types=[
            pltpu.VMEM((WR, 128), jnp.int32),
            pltpu.VMEM((WR, 128), jnp.int32),
            pltpu.VMEM((WR, 128), jnp.int32),
            pltpu.VMEM((WR, 128), jnp.float32),
            pltpu.VMEM((WR, 128), jnp.float32),
            pltpu.VMEM((WR, 128), jnp.float32),
            pltpu.VMEM((ZB,), jnp.float32),
            pltpu.VMEM_SHARED((CH,), jnp.float32),
            pltpu.SemaphoreType.DMA,
            pltpu.SemaphoreType.DMA,
            pltpu.SemaphoreType.DMA,
            pltpu.SemaphoreType.DMA,
            pltpu.SemaphoreType.DMA,
            pltpu.SemaphoreType.DMA,
            pltpu.SemaphoreType.DMA,
            pltpu.SemaphoreType.DMA,
            pltpu.SemaphoreType.DMA,
        ],
        compiler_params=_compiler_params(),
    )
    def scatter_add_kernel(idx_hbm, vals_hbm, out_hbm, ibuf0, ibuf1, ibuf2,
                           vbuf0, vbuf1, vbuf2, zbuf, shared, semi0, semi1,
                           semi2, semv0, semv1, semv2, sems0, sems1, sems2):
        core = lax.axis_index("c")
        sid = lax.axis_index("s")
        ibufs = (ibuf0, ibuf1, ibuf2)
        vbufs = (vbuf0, vbuf1, vbuf2)
        semis = (semi0, semi1, semi2)
        semvs = (semv0, semv1, semv2)
        semss = (sems0, sems1, sems2)

        @pl.loop(0, ZB, step=16)
        def _(i):
            zbuf[pl.ds(i, 16)] = jnp.zeros((16,), jnp.float32)

        @pl.loop(0, NZ)
        def _(z):
            pltpu.sync_copy(zbuf, shared.at[pl.ds(sid * SLICE + z * ZB, ZB)])

        def row_of(b, w):
            return b * ROWS_B + (sid + 16 * w) * WR

        def issue_load(b, w, k):
            rb = row_of(b, w)
            pltpu.async_copy(idx_hbm.at[pl.ds(rb, WR)], ibufs[k], semis[k])
            pltpu.async_copy(vals_hbm.at[pl.ds(rb, WR)], vbufs[k], semvs[k])

        def wait_load(b, w, k):
            rb = row_of(b, w)
            pltpu.make_async_copy(idx_hbm.at[pl.ds(rb, WR)], ibufs[k],
                                  semis[k]).wait()
            pltpu.make_async_copy(vals_hbm.at[pl.ds(rb, WR)], vbufs[k],
                                  semvs[k]).wait()

        def drain_scatters(k):
            @pl.loop(0, WR, step=8)
            def _(r0):
                for k2 in range(8):
                    pltpu.make_async_copy(
                        vbufs[k].at[r0 + k2],
                        shared.at[plsc.Indices(ibufs[k].at[r0 + k2],
                                               ignored_value=-1)],
                        semss[k],
                    ).wait()

        @pl.loop(0, CHUNKS // 2)
        def _(ci):
            cid = 2 * ci + core
            b = cid // NCH
            j = cid - b * NCH
            lo = j * CH
            nwin = jnp.where(sid < NWIN - 16 * (NWIN // 16), NWIN // 16 + 1,
                             NWIN // 16)
            plsc.subcore_barrier()

            issue_load(b, 0, 0)

            @pl.loop(0, ((NWIN // 16 + 1 + 2) // 3) * 3, step=3)
            def _(w0):
                for k in range(3):
                    w = w0 + k

                    @pl.when(w < nwin)
                    def _():
                        @pl.when(w >= 2)
                        def _():
                            drain_scatters((k + 1) % 3)

                        @pl.when(w + 1 < nwin)
                        def _():
                            issue_load(b, w + 1, (k + 1) % 3)

                        wait_load(b, w, k)
                        ib = ibufs[k]
                        vb = vbufs[k]

                        @pl.loop(0, WR)
                        def _(r):
                            row = ib.at[r]
                            for cc in range(8):
                                sl = pl.ds(cc * 16, 16)
                                v = row[sl]
                                t = v - lo
                                m = (t >= 0) & (t < CH)
                                row[sl] = jnp.where(m, t, -1)

                        @pl.loop(0, WR, step=8)
                        def _(r0):
                            for k2 in range(8):
                                pltpu.async_copy(
                                    vb.at[r0 + k2],
                                    shared.at[plsc.Indices(
                                        ib.at[r0 + k2], ignored_value=-1)],
                                    semss[k],
                                    add=True,
                                )

            for k in range(3):
                @pl.when((lax.rem(nwin - 2, 3) == k)
                         | (lax.rem(nwin - 1, 3) == k))
                def _():
                    drain_scatters(k)

            plsc.subcore_barrier()
            pltpu.sync_copy(
                shared.at[pl.ds(sid * SLICE, SLICE)],
                out_hbm.at[pl.ds(b * P + lo + sid * SLICE, SLICE)],
            )

            @pl.loop(0, NZ)
            def _(z):
                pltpu.sync_copy(zbuf,
                                shared.at[pl.ds(sid * SLICE + z * ZB, ZB)])

    out = scatter_add_kernel(idx, vals)
    return out.reshape(B, H_OUT, W_OUT, C)
